# baseline (speedup 1.0000x reference)
import functools

import jax
import jax.numpy as jnp
from jax import lax
from jax.experimental import pallas as pl
from jax.experimental.pallas import tpu as pltpu

N_NODES = 10000


def _mm_body(x_ref, w_ref, o_ref):
    o_ref[...] = jnp.dot(x_ref[...], w_ref[...],
                         preferred_element_type=jnp.float32)


def _matmul(x, w, m_tile=2000):
    m, k = x.shape
    k2, n = w.shape
    assert k == k2 and m % m_tile == 0
    grid = (m // m_tile,)
    return pl.pallas_call(
        _mm_body,
        grid=grid,
        in_specs=[
            pl.BlockSpec((m_tile, k), lambda i: (i, 0)),
            pl.BlockSpec((k, n), lambda i: (0, 0)),
        ],
        out_specs=pl.BlockSpec((m_tile, n), lambda i: (i, 0)),
        out_shape=jax.ShapeDtypeStruct((m, n), jnp.float32),
    )(x, w)


def _gat_layer(x, src, dst, W, a_src, a_dst, b, heads, out_dim, concat):
    n = x.shape[0]
    f_in = x.shape[1]
    hd = heads * out_dim
    hd_pad = ((hd + 127) // 128) * 128
    if hd_pad != hd:
        Wp = jnp.zeros((f_in, hd_pad), jnp.float32).at[:, :hd].set(W)
    else:
        Wp = W
    h = _matmul(x, Wp)[:, :hd]

    hr = h.reshape(n, heads, out_dim)
    alpha_s = (hr * a_src[None, :, :]).sum(-1)
    alpha_d = (hr * a_dst[None, :, :]).sum(-1)
    e = alpha_s[src] + alpha_d[dst]
    e = jnp.where(e >= 0, e, 0.2 * e)
    e_max = jax.ops.segment_max(e, dst, num_segments=n)
    e_exp = jnp.exp(e - e_max[dst])
    denom = jax.ops.segment_sum(e_exp, dst, num_segments=n)
    alpha = e_exp / (denom[dst] + 1e-16)
    msg = hr[src] * alpha[:, :, None]
    out = jax.ops.segment_sum(msg, dst, num_segments=n)
    if concat:
        out = out.reshape(n, hd)
    else:
        out = out.mean(axis=1)
    return out + b


def kernel(x, edge_index, W1, a1_src, a1_dst, b1, W2, a2_src, a2_dst, b2,
           W3, a3_src, a3_dst, b3):
    n = x.shape[0]
    loop = jnp.arange(n, dtype=edge_index.dtype)
    src = jnp.concatenate([edge_index[0], loop])
    dst = jnp.concatenate([edge_index[1], loop])
    h = _gat_layer(x, src, dst, W1, a1_src, a1_dst, b1, 4, 256, True)
    h = jax.nn.elu(h)
    h = _gat_layer(h, src, dst, W2, a2_src, a2_dst, b2, 4, 256, True)
    h = jax.nn.elu(h)
    h = _gat_layer(h, src, dst, W3, a3_src, a3_dst, b3, 6, 121, False)
    return jax.nn.sigmoid(h)


# speedup vs baseline: 3.3126x; 3.3126x over previous
import jax
import jax.numpy as jnp
from jax import lax
from jax.experimental import pallas as pl

N = 10000
NPAD = 10240
E_TOT = 170000
EPAD = 170240
TILE = 128
KC = 2048


def _mm_body(x_ref, w_ref, o_ref):
    o_ref[...] = jnp.dot(x_ref[...], w_ref[...],
                         preferred_element_type=jnp.float32)


def _matmul(x, w, m_tile=2048):
    m, k = x.shape
    _, n = w.shape
    return pl.pallas_call(
        _mm_body,
        grid=(m // m_tile,),
        in_specs=[
            pl.BlockSpec((m_tile, k), lambda i: (i, 0)),
            pl.BlockSpec((k, n), lambda i: (0, 0)),
        ],
        out_specs=pl.BlockSpec((m_tile, n), lambda i: (i, 0)),
        out_shape=jax.ShapeDtypeStruct((m, n), jnp.float32),
    )(x, w)


def _s_body(a_ref, d_ref, o_ref):
    e = a_ref[...] + d_ref[...]
    e = jnp.where(e >= 0, e, 0.2 * e)
    o_ref[...] = jnp.exp(e)


def _edge_weights(asT, adT):
    H = asT.shape[0]
    bl = EPAD // 10
    return pl.pallas_call(
        _s_body,
        grid=(10,),
        in_specs=[
            pl.BlockSpec((H, bl), lambda i: (0, i)),
            pl.BlockSpec((H, bl), lambda i: (0, i)),
        ],
        out_specs=pl.BlockSpec((H, bl), lambda i: (0, i)),
        out_shape=jax.ShapeDtypeStruct((H, EPAD), jnp.float32),
    )(asT, adT)


def _smm_body(s_ref, h_ref, o_ref):
    k = pl.program_id(2)

    @pl.when(k == 0)
    def _():
        o_ref[...] = jnp.zeros_like(o_ref)

    o_ref[...] += jnp.dot(s_ref[0], h_ref[0],
                          preferred_element_type=jnp.float32)[None]


def _attend(S, haug):
    H, _, DA = haug.shape
    return pl.pallas_call(
        _smm_body,
        grid=(H, NPAD // TILE, NPAD // KC),
        in_specs=[
            pl.BlockSpec((1, TILE, KC), lambda h, t, k: (h, t, k)),
            pl.BlockSpec((1, KC, DA), lambda h, t, k: (h, k, 0)),
        ],
        out_specs=pl.BlockSpec((1, TILE, DA), lambda h, t, k: (h, t, 0)),
        out_shape=jax.ShapeDtypeStruct((H, NPAD, DA), jnp.float32),
    )(S, haug)


def _norm12_body(mm_ref, b_ref, o_ref):
    mm = mm_ref[0]
    num = mm[:, :256]
    den = mm[:, 256][:, None]
    v = num / (den + 1e-16) + b_ref[pl.program_id(0)]
    o_ref[0] = jnp.where(v >= 0, v, jnp.exp(v) - 1.0)


def _normalize12(mm, b):
    H = mm.shape[0]
    return pl.pallas_call(
        _norm12_body,
        grid=(H, NPAD // TILE),
        in_specs=[
            pl.BlockSpec((1, TILE, 384), lambda h, t: (h, t, 0)),
            pl.BlockSpec((H, 256), lambda h, t: (0, 0)),
        ],
        out_specs=pl.BlockSpec((1, TILE, 256), lambda h, t: (h, t, 0)),
        out_shape=jax.ShapeDtypeStruct((H, NPAD, 256), jnp.float32),
    )(mm, b.reshape(H, 256))


def _norm3_body(mm_ref, b_ref, o_ref):
    acc = jnp.zeros((TILE, 128), jnp.float32)
    for h in range(6):
        num = mm_ref[h, :, :128]
        den = mm_ref[h, :, 128][:, None]
        acc = acc + num / (den + 1e-16)
    v = acc * (1.0 / 6.0) + b_ref[0]
    o_ref[...] = 1.0 / (1.0 + jnp.exp(-v))


def _normalize3(mm, b128):
    return pl.pallas_call(
        _norm3_body,
        grid=(NPAD // TILE,),
        in_specs=[
            pl.BlockSpec((6, TILE, 256), lambda t: (0, t, 0)),
            pl.BlockSpec((1, 128), lambda t: (0, 0)),
        ],
        out_specs=pl.BlockSpec((TILE, 128), lambda t: (t, 0)),
        out_shape=jax.ShapeDtypeStruct((NPAD, 128), jnp.float32),
    )(mm, b128.reshape(1, 128))


def _layer_weights(W, a_src, a_dst, hd_pad, alpha_pad=128):
    f_in, hd = W.shape
    H, D = a_src.shape
    Wh = W.reshape(f_in, H, D)
    was = jnp.einsum("fhd,hd->fh", Wh, a_src)
    wad = jnp.einsum("fhd,hd->fh", Wh, a_dst)
    ablk = jnp.zeros((f_in, alpha_pad), jnp.float32)
    ablk = ablk.at[:, :H].set(was).at[:, H:2 * H].set(wad)
    if hd_pad != hd:
        Wp = jnp.zeros((f_in, hd_pad), jnp.float32)
        for h in range(H):
            Wp = Wp.at[:, h * (hd_pad // H):h * (hd_pad // H) + D].set(
                W[:, h * D:(h + 1) * D])
    else:
        Wp = W
    return jnp.concatenate([Wp, ablk], axis=1)


def _gat_layer(xin, W, a_src, a_dst, b, src, dst, hd_pad, final):
    H, D = a_src.shape
    Dh = hd_pad // H
    Wcat = _layer_weights(W, a_src, a_dst, hd_pad)
    hext = _matmul(xin, Wcat)

    as_n = hext[:, hd_pad:hd_pad + H]
    ad_n = hext[:, hd_pad + H:hd_pad + 2 * H]
    asT = jnp.zeros((H, EPAD), jnp.float32).at[:, :E_TOT].set(
        jnp.take(as_n, src, axis=0).T)
    adT = jnp.zeros((H, EPAD), jnp.float32).at[:, :E_TOT].set(
        jnp.take(ad_n, dst, axis=0).T)
    sT = _edge_weights(asT, adT)[:, :E_TOT]

    S = jnp.zeros((H, NPAD, NPAD), jnp.float32)
    S = S.at[:, dst, src].add(sT)

    DA = 384 if not final else 256
    h = hext[:, :hd_pad].reshape(NPAD, H, Dh).transpose(1, 0, 2)
    haug = jnp.zeros((H, NPAD, DA), jnp.float32)
    haug = haug.at[:, :, :Dh].set(h).at[:, :, 256 if not final else 128].set(1.0)

    mm = _attend(S, haug)

    if final:
        b128 = jnp.zeros((128,), jnp.float32).at[:b.shape[0]].set(b)
        return _normalize3(mm, b128)
    out = _normalize12(mm, b)
    return out.transpose(1, 0, 2).reshape(NPAD, hd_pad)


def kernel(x, edge_index, W1, a1_src, a1_dst, b1, W2, a2_src, a2_dst, b2,
           W3, a3_src, a3_dst, b3):
    loop = jnp.arange(N, dtype=edge_index.dtype)
    src = jnp.concatenate([edge_index[0], loop])
    dst = jnp.concatenate([edge_index[1], loop])
    dst, src = lax.sort((dst, src), num_keys=1)

    xp = jnp.zeros((NPAD, x.shape[1]), jnp.float32).at[:N].set(x)
    h1 = _gat_layer(xp, W1, a1_src, a1_dst, b1, src, dst, 1024, False)
    h2 = _gat_layer(h1, W2, a2_src, a2_dst, b2, src, dst, 1024, False)
    y = _gat_layer(h2, W3, a3_src, a3_dst, b3, src, dst, 768, True)
    return y[:N, :121]
